# baseline (speedup 1.0000x reference)
import functools

import jax
import jax.numpy as jnp
from jax import lax
from jax.experimental import pallas as pl
from jax.experimental.pallas import tpu as pltpu
from jax.experimental.pallas import tpu_sc as plsc

_F32 = jnp.float32


def _segment_sum_sc(table, edges_lin, zeros_blk):
    n, h = table.shape
    e = edges_lin.shape[0] // 2
    ncores, nsub = 2, 16
    nw = ncores * nsub
    epw = e // nw
    chunk = 1000
    nch = epw // chunk
    assert epw * nw == e and nch * chunk == epw and chunk % 8 == 0
    zrows = zeros_blk.shape[0]
    nzch = n // zrows
    assert nzch * zrows == n and zrows % 8 == 0
    zsteps = -(-nzch // nsub)

    mesh = plsc.VectorSubcoreMesh(core_axis_name="c", subcore_axis_name="s")

    @functools.partial(
        pl.kernel,
        out_type=jax.ShapeDtypeStruct((ncores, n, h), _F32),
        mesh=mesh,
        scratch_types=[
            pltpu.VMEM((nch, chunk), jnp.int32),
            pltpu.VMEM((nch, chunk), jnp.int32),
            pltpu.VMEM((2, chunk, h), _F32),
            pltpu.VMEM_SHARED((n, h), _F32),
            pltpu.SemaphoreType.DMA,
            pltpu.SemaphoreType.DMA,
            pltpu.SemaphoreType.DMA,
            pltpu.SemaphoreType.DMA,
            pltpu.SemaphoreType.DMA,
        ],
        compiler_params=pltpu.CompilerParams(use_tc_tiling_on_sc=False),
    )
    def seg_sum(table_hbm, edges_hbm, zeros_hbm, out_hbm,
                src_a, dst_a, rows_v, acc, gsem0, gsem1, ssem0, ssem1, isem):
        c = lax.axis_index("c")
        s = lax.axis_index("s")
        wid = s * ncores + c
        gsem = (gsem0, gsem1)
        ssem = (ssem0, ssem1)

        base_w = wid * epw
        idx_copies = []
        for t in range(nch):
            idx_copies.append(pltpu.async_copy(
                edges_hbm.at[pl.ds(base_w + t * chunk, chunk)],
                src_a.at[t], isem))
            idx_copies.append(pltpu.async_copy(
                edges_hbm.at[pl.ds(e + base_w + t * chunk, chunk)],
                dst_a.at[t], isem))

        for j0 in range(zsteps):
            j = s + nsub * j0
            @pl.when(j < nzch)
            def _():
                sl = pl.ds(j * zrows, zrows)
                pltpu.sync_copy(zeros_hbm, acc.at[sl])
        for cp in idx_copies:
            cp.wait()
        plsc.subcore_barrier()

        gathers = [pltpu.async_copy(table_hbm.at[src_a.at[0]], rows_v.at[0],
                                    gsem[0]), None]
        scatters = [None, None]
        for t in range(nch):
            b = t % 2
            gathers[b].wait()
            scatters[b] = pltpu.async_copy(
                rows_v.at[b], acc.at[dst_a.at[t]], ssem[b], add=True)
            if t + 1 < nch:
                nb = (t + 1) % 2
                if scatters[nb] is not None:
                    scatters[nb].wait()
                gathers[nb] = pltpu.async_copy(
                    table_hbm.at[src_a.at[t + 1]], rows_v.at[nb], gsem[nb])
        for b in range(2):
            if scatters[b] is not None:
                scatters[b].wait()

        plsc.subcore_barrier()
        for j0 in range(zsteps):
            j = s + nsub * j0
            @pl.when(j < nzch)
            def _():
                sl = pl.ds(j * zrows, zrows)
                pltpu.sync_copy(acc.at[sl], out_hbm.at[c, sl])

    return seg_sum(table, edges_lin, zeros_blk)


def _proj_packed(x3, edge_index, w1):
    n8 = x3.shape[0]
    rb = n8
    d = x3.shape[2]
    h = w1.shape[1]
    e = edge_index.shape[1]

    def body(x_ref, e_ref, w_ref, o_ref, el_ref):
        xs = x_ref[...]
        w = w_ref[...]
        ys = [jnp.dot(xs[:, a, :], w, preferred_element_type=_F32)
              for a in range(8)]
        o_ref[...] = jnp.concatenate(ys, axis=1)
        el_ref[...] = jnp.concatenate([e_ref[0], e_ref[1]])

    return pl.pallas_call(
        body,
        grid=(n8 // rb,),
        in_specs=[pl.BlockSpec((rb, 8, d), lambda i: (i, 0, 0)),
                  pl.BlockSpec((2, e), lambda i: (0, 0)),
                  pl.BlockSpec((d, h), lambda i: (0, 0))],
        out_specs=[pl.BlockSpec((rb, 8 * h), lambda i: (i, 0)),
                   pl.BlockSpec((2 * e,), lambda i: (0,))],
        out_shape=[jax.ShapeDtypeStruct((n8, 8 * h), _F32),
                   jax.ShapeDtypeStruct((2 * e,), jnp.int32)],
    )(x3, edge_index, w1)


def _mlp_mid_packed(y, parts, b1t, w2bd, b2t):
    n8, hp = y.shape
    rb = n8

    def body(y_ref, p_ref, b1_ref, w2_ref, b2_ref, o_ref):
        t = jnp.maximum(y_ref[...] + p_ref[0] + p_ref[1] + b1_ref[...], 0.0)
        u = jnp.dot(t, w2_ref[...], preferred_element_type=_F32) + b2_ref[...]
        o_ref[...] = jnp.maximum(u, 0.0)

    return pl.pallas_call(
        body,
        grid=(n8 // rb,),
        in_specs=[pl.BlockSpec((rb, hp), lambda i: (i, 0)),
                  pl.BlockSpec((2, rb, hp), lambda i: (0, i, 0)),
                  pl.BlockSpec((1, hp), lambda i: (0, 0)),
                  pl.BlockSpec((hp, hp), lambda i: (0, 0)),
                  pl.BlockSpec((1, hp), lambda i: (0, 0))],
        out_specs=pl.BlockSpec((rb, hp), lambda i: (i, 0)),
        out_shape=jax.ShapeDtypeStruct((n8, hp), _F32),
    )(y, parts, b1t.reshape(1, hp), w2bd, b2t.reshape(1, hp))


def _mlp_out_packed(hh, parts, w3bd, b3t, w4cat, b4cat):
    n8, hp = hh.shape
    rb = n8
    o = w4cat.shape[1] // 8

    def body(h_ref, p_ref, w3_ref, b3_ref, w4_ref, b4_ref, o_ref):
        g = h_ref[...] + p_ref[0] + p_ref[1]
        t = jnp.maximum(jnp.dot(g, w3_ref[...], preferred_element_type=_F32)
                        + b3_ref[...], 0.0)
        z = jnp.dot(t, w4_ref[...], preferred_element_type=_F32) + b4_ref[...]
        for a in range(8):
            za = z[:, a * o:(a + 1) * o]
            m = jnp.max(za, axis=1, keepdims=True)
            lse = jnp.log(jnp.sum(jnp.exp(za - m), axis=1, keepdims=True)) + m
            o_ref[:, a, :] = za - lse

    return pl.pallas_call(
        body,
        grid=(n8 // rb,),
        in_specs=[pl.BlockSpec((rb, hp), lambda i: (i, 0)),
                  pl.BlockSpec((2, rb, hp), lambda i: (0, i, 0)),
                  pl.BlockSpec((hp, hp), lambda i: (0, 0)),
                  pl.BlockSpec((1, hp), lambda i: (0, 0)),
                  pl.BlockSpec((hp, 8 * o), lambda i: (0, 0)),
                  pl.BlockSpec((1, 8 * o), lambda i: (0, 0))],
        out_specs=pl.BlockSpec((rb, 8, o), lambda i: (i, 0, 0)),
        out_shape=jax.ShapeDtypeStruct((n8, 8, o), _F32),
    )(hh, parts, w3bd, b3t.reshape(1, hp), w4cat, b4cat.reshape(1, 8 * o))


def kernel(x, edge_index, W1, b1, W2, b2, W3, b3, W4, b4):
    n, d = x.shape
    h = W1.shape[1]
    n8 = n // 8
    zeros_blk = jnp.zeros((1000, h), _F32)

    eye8 = jnp.eye(8, dtype=_F32)
    w2bd = jnp.kron(eye8, W2)
    w3bd = jnp.kron(eye8, W3)
    w4cat = jnp.kron(eye8, W4)
    b1t = jnp.tile(b1, 8)
    b2t = jnp.tile(b2, 8)
    b3t = jnp.tile(b3, 8)
    b4cat = jnp.tile(b4, 8)

    x3 = x.reshape(n8, 8, d)
    y, edges_lin = _proj_packed(x3, edge_index, W1)
    parts1 = _segment_sum_sc(y.reshape(n, h), edges_lin, zeros_blk)
    hh = _mlp_mid_packed(y, parts1.reshape(2, n8, 8 * h), b1t, w2bd, b2t)
    parts2 = _segment_sum_sc(hh.reshape(n, h), edges_lin, zeros_blk)
    out = _mlp_out_packed(hh, parts2.reshape(2, n8, 8 * h), w3bd, b3t,
                          w4cat, b4cat)
    return out.reshape(n, W4.shape[1])


# speedup vs baseline: 1.0598x; 1.0598x over previous
import functools

import jax
import jax.numpy as jnp
from jax import lax
from jax.experimental import pallas as pl
from jax.experimental.pallas import tpu as pltpu
from jax.experimental.pallas import tpu_sc as plsc

_F32 = jnp.float32


def _segment_sum_sc(table, edges_lin, zeros_blk):
    n, h = table.shape
    e = edges_lin.shape[0] // 2
    ncores, nsub = 2, 16
    nw = ncores * nsub
    epw = e // nw
    chunk = 2000
    nch = epw // chunk
    assert epw * nw == e and nch * chunk == epw and chunk % 8 == 0
    zrows = zeros_blk.shape[0]
    nzch = n // zrows
    assert nzch * zrows == n and zrows % 8 == 0
    zsteps = -(-nzch // nsub)

    mesh = plsc.VectorSubcoreMesh(core_axis_name="c", subcore_axis_name="s")

    @functools.partial(
        pl.kernel,
        out_type=jax.ShapeDtypeStruct((ncores, n, h), _F32),
        mesh=mesh,
        scratch_types=[
            pltpu.VMEM((nch, chunk), jnp.int32),
            pltpu.VMEM((nch, chunk), jnp.int32),
            pltpu.VMEM((3, chunk, h), _F32),
            pltpu.VMEM_SHARED((n, h), _F32),
            pltpu.SemaphoreType.DMA,
            pltpu.SemaphoreType.DMA,
            pltpu.SemaphoreType.DMA,
            pltpu.SemaphoreType.DMA,
            pltpu.SemaphoreType.DMA,
            pltpu.SemaphoreType.DMA,
            pltpu.SemaphoreType.DMA,
        ],
        compiler_params=pltpu.CompilerParams(use_tc_tiling_on_sc=False),
    )
    def seg_sum(table_hbm, edges_hbm, zeros_hbm, out_hbm,
                src_a, dst_a, rows_v, acc, gsem0, gsem1, gsem2,
                ssem0, ssem1, ssem2, isem):
        c = lax.axis_index("c")
        s = lax.axis_index("s")
        wid = s * ncores + c
        gsem = (gsem0, gsem1, gsem2)
        ssem = (ssem0, ssem1, ssem2)

        base_w = wid * epw
        idx_copies = []
        for t in range(nch):
            idx_copies.append(pltpu.async_copy(
                edges_hbm.at[pl.ds(base_w + t * chunk, chunk)],
                src_a.at[t], isem))
            idx_copies.append(pltpu.async_copy(
                edges_hbm.at[pl.ds(e + base_w + t * chunk, chunk)],
                dst_a.at[t], isem))

        for j0 in range(zsteps):
            j = s + nsub * j0
            @pl.when(j < nzch)
            def _():
                sl = pl.ds(j * zrows, zrows)
                pltpu.sync_copy(zeros_hbm, acc.at[sl])
        for cp in idx_copies:
            cp.wait()
        plsc.subcore_barrier()

        nbuf = 3
        gathers = [None] * nbuf
        scatters = [None] * nbuf
        for t in range(min(2, nch)):
            gathers[t % nbuf] = pltpu.async_copy(
                table_hbm.at[src_a.at[t]], rows_v.at[t % nbuf], gsem[t % nbuf])
        for t in range(nch):
            b = t % nbuf
            gathers[b].wait()
            scatters[b] = pltpu.async_copy(
                rows_v.at[b], acc.at[dst_a.at[t]], ssem[b], add=True)
            if t + 2 < nch:
                nb = (t + 2) % nbuf
                if scatters[nb] is not None:
                    scatters[nb].wait()
                gathers[nb] = pltpu.async_copy(
                    table_hbm.at[src_a.at[t + 2]], rows_v.at[nb], gsem[nb])
        for b in range(nbuf):
            if scatters[b] is not None:
                scatters[b].wait()

        plsc.subcore_barrier()
        for j0 in range(zsteps):
            j = s + nsub * j0
            @pl.when(j < nzch)
            def _():
                sl = pl.ds(j * zrows, zrows)
                pltpu.sync_copy(acc.at[sl], out_hbm.at[c, sl])

    return seg_sum(table, edges_lin, zeros_blk)


def _proj_packed(x3, edge_index, w1):
    n8 = x3.shape[0]
    rb = n8
    d = x3.shape[2]
    h = w1.shape[1]
    e = edge_index.shape[1]

    def body(x_ref, e_ref, w_ref, o_ref, el_ref):
        xs = x_ref[...]
        w = w_ref[...]
        ys = [jnp.dot(xs[:, a, :], w, preferred_element_type=_F32)
              for a in range(8)]
        o_ref[...] = jnp.concatenate(ys, axis=1)
        el_ref[...] = jnp.concatenate([e_ref[0], e_ref[1]])

    return pl.pallas_call(
        body,
        grid=(n8 // rb,),
        in_specs=[pl.BlockSpec((rb, 8, d), lambda i: (i, 0, 0)),
                  pl.BlockSpec((2, e), lambda i: (0, 0)),
                  pl.BlockSpec((d, h), lambda i: (0, 0))],
        out_specs=[pl.BlockSpec((rb, 8 * h), lambda i: (i, 0)),
                   pl.BlockSpec((2 * e,), lambda i: (0,))],
        out_shape=[jax.ShapeDtypeStruct((n8, 8 * h), _F32),
                   jax.ShapeDtypeStruct((2 * e,), jnp.int32)],
    )(x3, edge_index, w1)


def _mlp_mid_packed(y, parts, b1t, w2bd, b2t):
    n8, hp = y.shape
    rb = n8

    def body(y_ref, p_ref, b1_ref, w2_ref, b2_ref, o_ref):
        t = jnp.maximum(y_ref[...] + p_ref[0] + p_ref[1] + b1_ref[...], 0.0)
        u = jnp.dot(t, w2_ref[...], preferred_element_type=_F32) + b2_ref[...]
        o_ref[...] = jnp.maximum(u, 0.0)

    return pl.pallas_call(
        body,
        grid=(n8 // rb,),
        in_specs=[pl.BlockSpec((rb, hp), lambda i: (i, 0)),
                  pl.BlockSpec((2, rb, hp), lambda i: (0, i, 0)),
                  pl.BlockSpec((1, hp), lambda i: (0, 0)),
                  pl.BlockSpec((hp, hp), lambda i: (0, 0)),
                  pl.BlockSpec((1, hp), lambda i: (0, 0))],
        out_specs=pl.BlockSpec((rb, hp), lambda i: (i, 0)),
        out_shape=jax.ShapeDtypeStruct((n8, hp), _F32),
    )(y, parts, b1t.reshape(1, hp), w2bd, b2t.reshape(1, hp))


def _mlp_out_packed(hh, parts, w3bd, b3t, w4cat, b4cat):
    n8, hp = hh.shape
    rb = n8
    o = w4cat.shape[1] // 8

    def body(h_ref, p_ref, w3_ref, b3_ref, w4_ref, b4_ref, o_ref):
        g = h_ref[...] + p_ref[0] + p_ref[1]
        t = jnp.maximum(jnp.dot(g, w3_ref[...], preferred_element_type=_F32)
                        + b3_ref[...], 0.0)
        z = jnp.dot(t, w4_ref[...], preferred_element_type=_F32) + b4_ref[...]
        for a in range(8):
            za = z[:, a * o:(a + 1) * o]
            m = jnp.max(za, axis=1, keepdims=True)
            lse = jnp.log(jnp.sum(jnp.exp(za - m), axis=1, keepdims=True)) + m
            o_ref[:, a, :] = za - lse

    return pl.pallas_call(
        body,
        grid=(n8 // rb,),
        in_specs=[pl.BlockSpec((rb, hp), lambda i: (i, 0)),
                  pl.BlockSpec((2, rb, hp), lambda i: (0, i, 0)),
                  pl.BlockSpec((hp, hp), lambda i: (0, 0)),
                  pl.BlockSpec((1, hp), lambda i: (0, 0)),
                  pl.BlockSpec((hp, 8 * o), lambda i: (0, 0)),
                  pl.BlockSpec((1, 8 * o), lambda i: (0, 0))],
        out_specs=pl.BlockSpec((rb, 8, o), lambda i: (i, 0, 0)),
        out_shape=jax.ShapeDtypeStruct((n8, 8, o), _F32),
    )(hh, parts, w3bd, b3t.reshape(1, hp), w4cat, b4cat.reshape(1, 8 * o))


def kernel(x, edge_index, W1, b1, W2, b2, W3, b3, W4, b4):
    n, d = x.shape
    h = W1.shape[1]
    n8 = n // 8
    zeros_blk = jnp.zeros((1000, h), _F32)

    eye8 = jnp.eye(8, dtype=_F32)
    w2bd = jnp.kron(eye8, W2)
    w3bd = jnp.kron(eye8, W3)
    w4cat = jnp.kron(eye8, W4)
    b1t = jnp.tile(b1, 8)
    b2t = jnp.tile(b2, 8)
    b3t = jnp.tile(b3, 8)
    b4cat = jnp.tile(b4, 8)

    x3 = x.reshape(n8, 8, d)
    y, edges_lin = _proj_packed(x3, edge_index, W1)
    parts1 = _segment_sum_sc(y.reshape(n, h), edges_lin, zeros_blk)
    hh = _mlp_mid_packed(y, parts1.reshape(2, n8, 8 * h), b1t, w2bd, b2t)
    parts2 = _segment_sum_sc(hh.reshape(n, h), edges_lin, zeros_blk)
    out = _mlp_out_packed(hh, parts2.reshape(2, n8, 8 * h), w3bd, b3t,
                          w4cat, b4cat)
    return out.reshape(n, W4.shape[1])


# speedup vs baseline: 1.0728x; 1.0123x over previous
import functools

import jax
import jax.numpy as jnp
from jax import lax
from jax.experimental import pallas as pl
from jax.experimental.pallas import tpu as pltpu
from jax.experimental.pallas import tpu_sc as plsc

_F32 = jnp.float32


def _segment_sum_sc(table, edges_lin, zeros_blk):
    n, h = table.shape
    e = edges_lin.shape[0] // 2
    ncores, nsub = 2, 16
    nw = ncores * nsub
    epw = e // nw
    chunk = 2000
    nch = epw // chunk
    assert epw * nw == e and nch * chunk == epw and chunk % 8 == 0
    zrows = zeros_blk.shape[0]
    nzch = n // zrows
    assert nzch * zrows == n and zrows % 8 == 0
    zsteps = -(-nzch // nsub)

    mesh = plsc.VectorSubcoreMesh(core_axis_name="c", subcore_axis_name="s")

    @functools.partial(
        pl.kernel,
        out_type=jax.ShapeDtypeStruct((ncores, n, h), _F32),
        mesh=mesh,
        scratch_types=[
            pltpu.VMEM((nch, chunk), jnp.int32),
            pltpu.VMEM((nch, chunk), jnp.int32),
            pltpu.VMEM((2, chunk, h), _F32),
            pltpu.VMEM_SHARED((n, h), _F32),
            pltpu.SemaphoreType.DMA,
            pltpu.SemaphoreType.DMA,
            pltpu.SemaphoreType.DMA,
            pltpu.SemaphoreType.DMA,
            pltpu.SemaphoreType.DMA,
        ],
        compiler_params=pltpu.CompilerParams(use_tc_tiling_on_sc=False),
    )
    def seg_sum(table_hbm, edges_hbm, zeros_hbm, out_hbm,
                src_a, dst_a, rows_v, acc, gsem0, gsem1, ssem0, ssem1, isem):
        c = lax.axis_index("c")
        s = lax.axis_index("s")
        wid = s * ncores + c
        gsem = (gsem0, gsem1)
        ssem = (ssem0, ssem1)

        base_w = wid * epw
        idx_copies = []
        for t in range(nch):
            idx_copies.append(pltpu.async_copy(
                edges_hbm.at[pl.ds(base_w + t * chunk, chunk)],
                src_a.at[t], isem))
            idx_copies.append(pltpu.async_copy(
                edges_hbm.at[pl.ds(e + base_w + t * chunk, chunk)],
                dst_a.at[t], isem))

        for j0 in range(zsteps):
            j = s + nsub * j0
            @pl.when(j < nzch)
            def _():
                sl = pl.ds(j * zrows, zrows)
                pltpu.sync_copy(zeros_hbm, acc.at[sl])
        for cp in idx_copies:
            cp.wait()
        plsc.subcore_barrier()

        gathers = [pltpu.async_copy(table_hbm.at[src_a.at[0]], rows_v.at[0],
                                    gsem[0]), None]
        scatters = [None, None]
        for t in range(nch):
            b = t % 2
            gathers[b].wait()
            scatters[b] = pltpu.async_copy(
                rows_v.at[b], acc.at[dst_a.at[t]], ssem[b], add=True)
            if t + 1 < nch:
                nb = (t + 1) % 2
                if scatters[nb] is not None:
                    scatters[nb].wait()
                gathers[nb] = pltpu.async_copy(
                    table_hbm.at[src_a.at[t + 1]], rows_v.at[nb], gsem[nb])
        for b in range(2):
            if scatters[b] is not None:
                scatters[b].wait()

        plsc.subcore_barrier()
        for j0 in range(zsteps):
            j = s + nsub * j0
            @pl.when(j < nzch)
            def _():
                sl = pl.ds(j * zrows, zrows)
                pltpu.sync_copy(acc.at[sl], out_hbm.at[c, sl])

    return seg_sum(table, edges_lin, zeros_blk)


def _proj_packed(x3, edge_index, w1):
    n8 = x3.shape[0]
    rb = n8
    d = x3.shape[2]
    h = w1.shape[1]
    e = edge_index.shape[1]

    def body(x_ref, e_ref, w_ref, o_ref, el_ref):
        xs = x_ref[...]
        w = w_ref[...]
        ys = [jnp.dot(xs[:, a, :], w, preferred_element_type=_F32)
              for a in range(8)]
        o_ref[...] = jnp.concatenate(ys, axis=1)
        el_ref[...] = jnp.concatenate([e_ref[0], e_ref[1]])

    return pl.pallas_call(
        body,
        grid=(n8 // rb,),
        in_specs=[pl.BlockSpec((rb, 8, d), lambda i: (i, 0, 0)),
                  pl.BlockSpec((2, e), lambda i: (0, 0)),
                  pl.BlockSpec((d, h), lambda i: (0, 0))],
        out_specs=[pl.BlockSpec((rb, 8 * h), lambda i: (i, 0)),
                   pl.BlockSpec((2 * e,), lambda i: (0,))],
        out_shape=[jax.ShapeDtypeStruct((n8, 8 * h), _F32),
                   jax.ShapeDtypeStruct((2 * e,), jnp.int32)],
    )(x3, edge_index, w1)


def _mlp_mid_packed(y, parts, b1t, w2bd, b2t):
    n8, hp = y.shape
    rb = n8

    def body(y_ref, p_ref, b1_ref, w2_ref, b2_ref, o_ref):
        t = jnp.maximum(y_ref[...] + p_ref[0] + p_ref[1] + b1_ref[...], 0.0)
        u = jnp.dot(t, w2_ref[...], preferred_element_type=_F32) + b2_ref[...]
        o_ref[...] = jnp.maximum(u, 0.0)

    return pl.pallas_call(
        body,
        grid=(n8 // rb,),
        in_specs=[pl.BlockSpec((rb, hp), lambda i: (i, 0)),
                  pl.BlockSpec((2, rb, hp), lambda i: (0, i, 0)),
                  pl.BlockSpec((1, hp), lambda i: (0, 0)),
                  pl.BlockSpec((hp, hp), lambda i: (0, 0)),
                  pl.BlockSpec((1, hp), lambda i: (0, 0))],
        out_specs=pl.BlockSpec((rb, hp), lambda i: (i, 0)),
        out_shape=jax.ShapeDtypeStruct((n8, hp), _F32),
    )(y, parts, b1t.reshape(1, hp), w2bd, b2t.reshape(1, hp))


def _mlp_out_packed(hh, parts, w3bd, b3t, w4cat, b4cat):
    n8, hp = hh.shape
    rb = n8
    o = w4cat.shape[1] // 8

    def body(h_ref, p_ref, w3_ref, b3_ref, w4_ref, b4_ref, o_ref):
        g = h_ref[...] + p_ref[0] + p_ref[1]
        t = jnp.maximum(jnp.dot(g, w3_ref[...], preferred_element_type=_F32)
                        + b3_ref[...], 0.0)
        z = jnp.dot(t, w4_ref[...], preferred_element_type=_F32) + b4_ref[...]
        for a in range(8):
            za = z[:, a * o:(a + 1) * o]
            m = jnp.max(za, axis=1, keepdims=True)
            lse = jnp.log(jnp.sum(jnp.exp(za - m), axis=1, keepdims=True)) + m
            o_ref[:, a, :] = za - lse

    return pl.pallas_call(
        body,
        grid=(n8 // rb,),
        in_specs=[pl.BlockSpec((rb, hp), lambda i: (i, 0)),
                  pl.BlockSpec((2, rb, hp), lambda i: (0, i, 0)),
                  pl.BlockSpec((hp, hp), lambda i: (0, 0)),
                  pl.BlockSpec((1, hp), lambda i: (0, 0)),
                  pl.BlockSpec((hp, 8 * o), lambda i: (0, 0)),
                  pl.BlockSpec((1, 8 * o), lambda i: (0, 0))],
        out_specs=pl.BlockSpec((rb, 8, o), lambda i: (i, 0, 0)),
        out_shape=jax.ShapeDtypeStruct((n8, 8, o), _F32),
    )(hh, parts, w3bd, b3t.reshape(1, hp), w4cat, b4cat.reshape(1, 8 * o))


def kernel(x, edge_index, W1, b1, W2, b2, W3, b3, W4, b4):
    n, d = x.shape
    h = W1.shape[1]
    n8 = n // 8
    zeros_blk = jnp.zeros((1000, h), _F32)

    eye8 = jnp.eye(8, dtype=_F32)
    w2bd = jnp.kron(eye8, W2)
    w3bd = jnp.kron(eye8, W3)
    w4cat = jnp.kron(eye8, W4)
    b1t = jnp.tile(b1, 8)
    b2t = jnp.tile(b2, 8)
    b3t = jnp.tile(b3, 8)
    b4cat = jnp.tile(b4, 8)

    x3 = x.reshape(n8, 8, d)
    y, edges_lin = _proj_packed(x3, edge_index, W1)
    parts1 = _segment_sum_sc(y.reshape(n, h), edges_lin, zeros_blk)
    hh = _mlp_mid_packed(y, parts1.reshape(2, n8, 8 * h), b1t, w2bd, b2t)
    parts2 = _segment_sum_sc(hh.reshape(n, h), edges_lin, zeros_blk)
    out = _mlp_out_packed(hh, parts2.reshape(2, n8, 8 * h), w3bd, b3t,
                          w4cat, b4cat)
    return out.reshape(n, W4.shape[1])
